# baseline (speedup 1.0000x reference)
import functools

import jax
import jax.numpy as jnp
from jax import lax
from jax.experimental import pallas as pl
from jax.experimental.pallas import tpu as pltpu
from jax.experimental.pallas import tpu_sc as plsc

N = 8192
K = 20
B = 8
BLK = 256
NBLK = N // BLK
BIG = 1e30
BIGI = 2**30
J = 4
KBLK = 64
NKBLK = N // KBLK


def _knn_body(f_rows_ref, f_allT_ref, sq_cT_ref, bat_col_ref, bat_rowT_ref,
              idx_ref):
    i = pl.program_id(0)
    f_rows = f_rows_ref[...]
    f_allT = f_allT_ref[...]
    mm = lax.dot_general(f_rows, f_allT, (((1,), (0,)), ((), ())),
                         preferred_element_type=jnp.float32)
    sq_r = jnp.sum(f_rows * f_rows, axis=1, keepdims=True)
    sq_c = sq_cT_ref[...]
    d = (sq_r + sq_c) - 2.0 * mm
    colio = lax.broadcasted_iota(jnp.int32, (KBLK, N), 1)
    rowio = lax.broadcasted_iota(jnp.int32, (KBLK, N), 0) + i * KBLK
    d = jnp.where(colio == rowio, BIG, d)
    d = jnp.where(bat_col_ref[...] != bat_rowT_ref[...], BIG, d)

    laneio = lax.broadcasted_iota(jnp.int32, (KBLK, 128), 1)
    L = [jnp.full((KBLK, 128), BIG, jnp.float32) for _ in range(J)]
    Cc = [jnp.full((KBLK, 128), BIGI, jnp.int32) for _ in range(J)]
    for g in range(N // 128):
        xv = d[:, g * 128:(g + 1) * 128]
        xc = laneio + (g * 128)
        for l in range(J):
            lt = xv < L[l]
            nv = jnp.minimum(xv, L[l])
            nc = jnp.where(lt, xc, Cc[l])
            dv = jnp.maximum(xv, L[l])
            dc = jnp.where(lt, Cc[l], xc)
            L[l], Cc[l] = nv, nc
            xv, xc = dv, dc

    cnt = jnp.zeros((KBLK, 128), jnp.int32)
    cols = []
    for _ in range(K):
        m = jnp.min(L[0], axis=1, keepdims=True)
        colstar = jnp.min(jnp.where(L[0] == m, Cc[0], BIGI), axis=1,
                          keepdims=True)
        mask = (L[0] == m) & (Cc[0] == colstar)
        cols.append(colstar)
        for l in range(J - 1):
            L[l] = jnp.where(mask, L[l + 1], L[l])
            Cc[l] = jnp.where(mask, Cc[l + 1], Cc[l])
        L[J - 1] = jnp.where(mask, BIG, L[J - 1])
        Cc[J - 1] = jnp.where(mask, BIGI, Cc[J - 1])
        cnt = cnt + mask.astype(jnp.int32)
    idx_ref[...] = jnp.concatenate(cols, axis=1)

    ov = jnp.max(jnp.where(cnt >= J, 1, 0))

    @pl.when(ov == 1)
    def _fallback():
        dd = d
        out = []
        for _ in range(K):
            mm = jnp.min(dd, axis=1, keepdims=True)
            cand = jnp.where(dd == mm, colio, BIGI)
            ik = jnp.min(cand, axis=1, keepdims=True)
            out.append(ik)
            dd = jnp.where(colio == ik, BIG, dd)
        idx_ref[...] = jnp.concatenate(out, axis=1)


def _colsq_body(fT_ref, out_ref):
    fT = fT_ref[...]
    out_ref[...] = jnp.sum(fT * fT, axis=0, keepdims=True)


def _knn(f, batch):
    F = f.shape[1]
    f_allT = f.T
    sq_cT = pl.pallas_call(
        _colsq_body,
        in_specs=[pl.BlockSpec((F, N), lambda: (0, 0))],
        out_specs=pl.BlockSpec((1, N), lambda: (0, 0)),
        out_shape=jax.ShapeDtypeStruct((1, N), jnp.float32),
    )(f_allT)
    bat_col = batch.reshape(N, 1)
    bat_rowT = batch.reshape(1, N)
    return pl.pallas_call(
        _knn_body,
        grid=(NKBLK,),
        in_specs=[
            pl.BlockSpec((KBLK, F), lambda i: (i, 0)),
            pl.BlockSpec((F, N), lambda i: (0, 0)),
            pl.BlockSpec((1, N), lambda i: (0, 0)),
            pl.BlockSpec((KBLK, 1), lambda i: (i, 0)),
            pl.BlockSpec((1, N), lambda i: (0, 0)),
        ],
        out_specs=pl.BlockSpec((KBLK, K), lambda i: (i, 0)),
        out_shape=jax.ShapeDtypeStruct((N, K), jnp.int32),
    )(f, f_allT, sq_cT, bat_col, bat_rowT)


def _sc_gather(table, idx_flat):
    C = table.shape[1]
    NW = 32
    tot = idx_flat.shape[0]
    b_per_w = tot // NW
    CH = 256
    n_ch = b_per_w // CH
    mesh = plsc.VectorSubcoreMesh(core_axis_name="c", subcore_axis_name="s")

    @functools.partial(
        pl.kernel, mesh=mesh,
        out_type=jax.ShapeDtypeStruct((tot, C), jnp.float32),
        scratch_types=[
            pltpu.VMEM((CH,), jnp.int32),
            pltpu.VMEM((CH, C), jnp.float32),
            pltpu.SemaphoreType.DMA,
        ],
    )
    def k(table_hbm, idx_hbm, out_hbm, idx_v, rows_v, sem):
        wid = lax.axis_index("s") * 2 + lax.axis_index("c")
        base = wid * b_per_w

        def body(j, carry):
            off = base + j * CH
            pltpu.sync_copy(idx_hbm.at[pl.ds(off, CH)], idx_v)
            pltpu.async_copy(table_hbm.at[idx_v], rows_v, sem).wait()
            pltpu.sync_copy(rows_v, out_hbm.at[pl.ds(off, CH)])
            return carry

        lax.fori_loop(0, n_ch, body, 0)

    return k(table, idx_flat)


def _edge_fin_body(g_ref, xi_ref, w_ref, b_ref, hmax_ref, s_ref, ss_ref, *, F):
    i = pl.program_id(0)
    k = pl.program_id(1)
    xi = xi_ref[...]
    xj = g_ref[:, :F]
    e = jnp.concatenate([xi, xj - xi], axis=1)
    h = lax.dot_general(e, w_ref[...], (((1,), (0,)), ((), ())),
                        preferred_element_type=jnp.float32) + b_ref[...]
    h = jnp.maximum(h, 0.0)
    s = jnp.sum(h, axis=0, keepdims=True)
    ss = jnp.sum(h * h, axis=0, keepdims=True)

    @pl.when(k == 0)
    def _init_max():
        hmax_ref[...] = h

    @pl.when(k > 0)
    def _acc_max():
        hmax_ref[...] = jnp.maximum(hmax_ref[...], h)

    @pl.when((i == 0) & (k == 0))
    def _init_s():
        s_ref[...] = s
        ss_ref[...] = ss

    @pl.when((i > 0) | (k > 0))
    def _acc_s():
        s_ref[...] = s_ref[...] + s
        ss_ref[...] = ss_ref[...] + ss


def _edge_finalize(gath_km, f, W, b):
    F = f.shape[1]
    C = W.shape[1]
    Cg = gath_km.shape[1]
    return pl.pallas_call(
        functools.partial(_edge_fin_body, F=F),
        grid=(NBLK, K),
        in_specs=[
            pl.BlockSpec((BLK, Cg), lambda i, k: (k * NBLK + i, 0)),
            pl.BlockSpec((BLK, F), lambda i, k: (i, 0)),
            pl.BlockSpec((2 * F, C), lambda i, k: (0, 0)),
            pl.BlockSpec((1, C), lambda i, k: (0, 0)),
        ],
        out_specs=[
            pl.BlockSpec((BLK, C), lambda i, k: (i, 0)),
            pl.BlockSpec((1, C), lambda i, k: (0, 0)),
            pl.BlockSpec((1, C), lambda i, k: (0, 0)),
        ],
        out_shape=[
            jax.ShapeDtypeStruct((N, C), jnp.float32),
            jax.ShapeDtypeStruct((1, C), jnp.float32),
            jax.ShapeDtypeStruct((1, C), jnp.float32),
        ],
    )(gath_km, f, W, b.reshape(1, C))


def _bn_apply_body(hmax_ref, s_ref, ss_ref, g_ref, be_ref, out_ref, *, cnt):
    m = s_ref[...] / cnt
    var = ss_ref[...] / cnt - m * m
    scale = g_ref[...] * lax.rsqrt(var + 1e-5)
    out_ref[...] = scale * (hmax_ref[...] - m) + be_ref[...]


def _bn_apply(hmax, s, ss, g, be, cnt):
    C = hmax.shape[1]
    return pl.pallas_call(
        functools.partial(_bn_apply_body, cnt=float(cnt)),
        grid=(NBLK,),
        in_specs=[
            pl.BlockSpec((BLK, C), lambda i: (i, 0)),
            pl.BlockSpec((1, C), lambda i: (0, 0)),
            pl.BlockSpec((1, C), lambda i: (0, 0)),
            pl.BlockSpec((1, C), lambda i: (0, 0)),
            pl.BlockSpec((1, C), lambda i: (0, 0)),
        ],
        out_specs=pl.BlockSpec((BLK, C), lambda i: (i, 0)),
        out_shape=jax.ShapeDtypeStruct((N, C), jnp.float32),
    )(hmax, s, ss, g.reshape(1, C), be.reshape(1, C))


def _fc_seg_body(xc_ref, wfc_ref, bfc_ref, bat_ref,
                 s_ref, ss_ref, seg_ref):
    i = pl.program_id(0)
    h = lax.dot_general(xc_ref[...], wfc_ref[...], (((1,), (0,)), ((), ())),
                        preferred_element_type=jnp.float32)
    h = jnp.maximum(h + bfc_ref[...], 0.0)
    s = jnp.sum(h, axis=0, keepdims=True)
    ss = jnp.sum(h * h, axis=0, keepdims=True)
    bat = bat_ref[...]
    segs = []
    for s_ in range(B):
        hm = jnp.where(bat == s_, h, -jnp.inf)
        segs.append(jnp.max(hm, axis=0, keepdims=True))
    seg = jnp.concatenate(segs, axis=0)

    @pl.when(i == 0)
    def _init():
        s_ref[...] = s
        ss_ref[...] = ss
        seg_ref[...] = seg

    @pl.when(i > 0)
    def _acc():
        s_ref[...] = s_ref[...] + s
        ss_ref[...] = ss_ref[...] + ss
        seg_ref[...] = jnp.maximum(seg_ref[...], seg)


def _fc_seg(xcat, wfc, bfc, batch):
    return pl.pallas_call(
        _fc_seg_body,
        grid=(NBLK,),
        in_specs=[
            pl.BlockSpec((BLK, 192), lambda i: (i, 0)),
            pl.BlockSpec((192, 256), lambda i: (0, 0)),
            pl.BlockSpec((1, 256), lambda i: (0, 0)),
            pl.BlockSpec((BLK, 1), lambda i: (i, 0)),
        ],
        out_specs=[
            pl.BlockSpec((1, 256), lambda i: (0, 0)),
            pl.BlockSpec((1, 256), lambda i: (0, 0)),
            pl.BlockSpec((B, 256), lambda i: (0, 0)),
        ],
        out_shape=[
            jax.ShapeDtypeStruct((1, 256), jnp.float32),
            jax.ShapeDtypeStruct((1, 256), jnp.float32),
            jax.ShapeDtypeStruct((B, 256), jnp.float32),
        ],
    )(xcat, wfc, bfc.reshape(1, 256), batch.reshape(N, 1))


def _head_body(s_ref, ss_ref, seg_ref, gfc_ref, befc_ref,
               wo1_ref, bo1_ref, go1_ref, beo1_ref,
               wo2_ref, bo2_ref, go2_ref, beo2_ref,
               wo3_ref, bo3_ref, out_ref):
    m = s_ref[...] / float(N)
    var = ss_ref[...] / float(N) - m * m
    p_raw = seg_ref[...]
    p = gfc_ref[...] * lax.rsqrt(var + 1e-5) * (p_raw - m) + befc_ref[...]

    def bn0(o, g, be):
        mm = jnp.mean(o, axis=0, keepdims=True)
        vv = jnp.mean(o * o, axis=0, keepdims=True) - mm * mm
        return g * lax.rsqrt(vv + 1e-5) * (o - mm) + be

    o = lax.dot_general(p, wo1_ref[...], (((1,), (0,)), ((), ())),
                        preferred_element_type=jnp.float32) + bo1_ref[...]
    o = bn0(jnp.maximum(o, 0.0), go1_ref[...], beo1_ref[...])
    o = lax.dot_general(o, wo2_ref[...], (((1,), (0,)), ((), ())),
                        preferred_element_type=jnp.float32) + bo2_ref[...]
    o = bn0(jnp.maximum(o, 0.0), go2_ref[...], beo2_ref[...])
    o = lax.dot_general(o, wo3_ref[...], (((1,), (0,)), ((), ())),
                        preferred_element_type=jnp.float32) + bo3_ref[...]
    mx = jnp.max(o, axis=1, keepdims=True)
    lse = jnp.log(jnp.sum(jnp.exp(o - mx), axis=1, keepdims=True))
    out_ref[...] = o - mx - lse


def _head(s, ss, seg, gfc, befc, wo1, bo1, go1, beo1, wo2, bo2, go2, beo2, wo3, bo3):
    full = lambda *shape: pl.BlockSpec(shape, lambda: tuple(0 for _ in shape))
    return pl.pallas_call(
        _head_body,
        in_specs=[
            full(1, 256), full(1, 256), full(B, 256),
            full(1, 256), full(1, 256),
            full(256, 128), full(1, 128), full(1, 128), full(1, 128),
            full(128, 64), full(1, 64), full(1, 64), full(1, 64),
            full(64, 40), full(1, 40),
        ],
        out_specs=full(B, 40),
        out_shape=jax.ShapeDtypeStruct((B, 40), jnp.float32),
    )(s, ss, seg, gfc.reshape(1, 256), befc.reshape(1, 256),
      wo1, bo1.reshape(1, 128), go1.reshape(1, 128), beo1.reshape(1, 128),
      wo2, bo2.reshape(1, 64), go2.reshape(1, 64), beo2.reshape(1, 64),
      wo3, bo3.reshape(1, 40))


def kernel(pos, x, batch, W1, b1, g1, be1, W2, b2, g2, be2, Wfc, bfc, gfc, befc,
           Wo1, bo1, go1, beo1, Wo2, bo2, go2, beo2, Wo3, bo3):
    batch = batch.astype(jnp.int32)
    f0 = jnp.concatenate([pos, x], axis=1)
    f0p8 = jnp.concatenate([f0, jnp.zeros((N, 4), jnp.float32)], axis=1)
    f0p128 = jnp.concatenate([f0, jnp.zeros((N, 124), jnp.float32)], axis=1)
    zeros_batch = jnp.zeros((N,), jnp.int32)

    idx1 = _knn(f0p8, zeros_batch)
    g1rows = _sc_gather(f0p128, idx1.T.reshape(N * K))
    hmax1, s1, ss1 = _edge_finalize(g1rows, f0, W1, b1)
    x1 = _bn_apply(hmax1, s1, ss1, g1, be1, N * K)

    idx2 = _knn(x1, batch)
    x1p128 = jnp.concatenate([x1, jnp.zeros((N, 64), jnp.float32)], axis=1)
    g2rows = _sc_gather(x1p128, idx2.T.reshape(N * K))
    hmax2, s2, ss2 = _edge_finalize(g2rows, x1, W2, b2)
    x2 = _bn_apply(hmax2, s2, ss2, g2, be2, N * K)

    sfc, ssfc, seg = _fc_seg(jnp.concatenate([x1, x2], axis=1), Wfc, bfc, batch)
    return _head(sfc, ssfc, seg, gfc, befc, Wo1, bo1, go1, beo1,
                 Wo2, bo2, go2, beo2, Wo3, bo3)


# speedup vs baseline: 1.1435x; 1.1435x over previous
import functools

import jax
import jax.numpy as jnp
from jax import lax
from jax.experimental import pallas as pl
from jax.experimental.pallas import tpu as pltpu
from jax.experimental.pallas import tpu_sc as plsc

N = 8192
K = 20
B = 8
BLK = 256
NBLK = N // BLK
BIG = 1e30
BIGI = 2**30
J = 4
KBLK = 128
NKBLK = N // KBLK


def _knn_body(f_rows_ref, f_allT_ref, sq_cT_ref, bat_col_ref, bat_rowT_ref,
              idx_ref):
    i = pl.program_id(0)
    f_rows = f_rows_ref[...]
    f_allT = f_allT_ref[...]
    mm = lax.dot_general(f_rows, f_allT, (((1,), (0,)), ((), ())),
                         preferred_element_type=jnp.float32)
    sq_r = jnp.sum(f_rows * f_rows, axis=1, keepdims=True)
    sq_c = sq_cT_ref[...]
    d = (sq_r + sq_c) - 2.0 * mm
    colio = lax.broadcasted_iota(jnp.int32, (KBLK, N), 1)
    rowio = lax.broadcasted_iota(jnp.int32, (KBLK, N), 0) + i * KBLK
    d = jnp.where(colio == rowio, BIG, d)
    d = jnp.where(bat_col_ref[...] != bat_rowT_ref[...], BIG, d)

    laneio = lax.broadcasted_iota(jnp.int32, (KBLK, 128), 1)
    L = [jnp.full((KBLK, 128), BIG, jnp.float32) for _ in range(J)]
    Cc = [jnp.full((KBLK, 128), BIGI, jnp.int32) for _ in range(J)]
    for g in range(N // 128):
        xv = d[:, g * 128:(g + 1) * 128]
        xc = laneio + (g * 128)
        for l in range(J):
            lt = xv < L[l]
            nv = jnp.minimum(xv, L[l])
            nc = jnp.where(lt, xc, Cc[l])
            dv = jnp.maximum(xv, L[l])
            dc = jnp.where(lt, Cc[l], xc)
            L[l], Cc[l] = nv, nc
            xv, xc = dv, dc

    cnt = jnp.zeros((KBLK, 128), jnp.int32)
    cols = []
    for _ in range(K):
        m = jnp.min(L[0], axis=1, keepdims=True)
        colstar = jnp.min(jnp.where(L[0] == m, Cc[0], BIGI), axis=1,
                          keepdims=True)
        mask = (L[0] == m) & (Cc[0] == colstar)
        cols.append(colstar)
        for l in range(J - 1):
            L[l] = jnp.where(mask, L[l + 1], L[l])
            Cc[l] = jnp.where(mask, Cc[l + 1], Cc[l])
        L[J - 1] = jnp.where(mask, BIG, L[J - 1])
        Cc[J - 1] = jnp.where(mask, BIGI, Cc[J - 1])
        cnt = cnt + mask.astype(jnp.int32)
    idx_ref[...] = jnp.concatenate(cols, axis=1)

    ov = jnp.max(jnp.where(cnt >= J, 1, 0))

    @pl.when(ov == 1)
    def _fallback():
        dd = d
        out = []
        for _ in range(K):
            mm = jnp.min(dd, axis=1, keepdims=True)
            cand = jnp.where(dd == mm, colio, BIGI)
            ik = jnp.min(cand, axis=1, keepdims=True)
            out.append(ik)
            dd = jnp.where(colio == ik, BIG, dd)
        idx_ref[...] = jnp.concatenate(out, axis=1)


def _colsq_body(fT_ref, out_ref):
    fT = fT_ref[...]
    out_ref[...] = jnp.sum(fT * fT, axis=0, keepdims=True)


def _knn(f, batch):
    F = f.shape[1]
    f_allT = f.T
    sq_cT = pl.pallas_call(
        _colsq_body,
        in_specs=[pl.BlockSpec((F, N), lambda: (0, 0))],
        out_specs=pl.BlockSpec((1, N), lambda: (0, 0)),
        out_shape=jax.ShapeDtypeStruct((1, N), jnp.float32),
    )(f_allT)
    bat_col = batch.reshape(N, 1)
    bat_rowT = batch.reshape(1, N)
    return pl.pallas_call(
        _knn_body,
        grid=(NKBLK,),
        in_specs=[
            pl.BlockSpec((KBLK, F), lambda i: (i, 0)),
            pl.BlockSpec((F, N), lambda i: (0, 0)),
            pl.BlockSpec((1, N), lambda i: (0, 0)),
            pl.BlockSpec((KBLK, 1), lambda i: (i, 0)),
            pl.BlockSpec((1, N), lambda i: (0, 0)),
        ],
        out_specs=pl.BlockSpec((KBLK, K), lambda i: (i, 0)),
        out_shape=jax.ShapeDtypeStruct((N, K), jnp.int32),
    )(f, f_allT, sq_cT, bat_col, bat_rowT)


def _sc_gather(table, idx_flat):
    C = table.shape[1]
    NW = 32
    tot = idx_flat.shape[0]
    b_per_w = tot // NW
    CH = 256
    n_ch = b_per_w // CH
    mesh = plsc.VectorSubcoreMesh(core_axis_name="c", subcore_axis_name="s")

    @functools.partial(
        pl.kernel, mesh=mesh,
        out_type=jax.ShapeDtypeStruct((tot, C), jnp.float32),
        scratch_types=[
            pltpu.VMEM((CH,), jnp.int32),
            pltpu.VMEM((CH, C), jnp.float32),
            pltpu.SemaphoreType.DMA,
        ],
    )
    def k(table_hbm, idx_hbm, out_hbm, idx_v, rows_v, sem):
        wid = lax.axis_index("s") * 2 + lax.axis_index("c")
        base = wid * b_per_w

        def body(j, carry):
            off = base + j * CH
            pltpu.sync_copy(idx_hbm.at[pl.ds(off, CH)], idx_v)
            pltpu.async_copy(table_hbm.at[idx_v], rows_v, sem).wait()
            pltpu.sync_copy(rows_v, out_hbm.at[pl.ds(off, CH)])
            return carry

        lax.fori_loop(0, n_ch, body, 0)

    return k(table, idx_flat)


def _edge_fin_body(g_ref, xi_ref, w_ref, b_ref, hmax_ref, s_ref, ss_ref, *, F):
    i = pl.program_id(0)
    k = pl.program_id(1)
    xi = xi_ref[...]
    xj = g_ref[:, :F]
    e = jnp.concatenate([xi, xj - xi], axis=1)
    h = lax.dot_general(e, w_ref[...], (((1,), (0,)), ((), ())),
                        preferred_element_type=jnp.float32) + b_ref[...]
    h = jnp.maximum(h, 0.0)
    s = jnp.sum(h, axis=0, keepdims=True)
    ss = jnp.sum(h * h, axis=0, keepdims=True)

    @pl.when(k == 0)
    def _init_max():
        hmax_ref[...] = h

    @pl.when(k > 0)
    def _acc_max():
        hmax_ref[...] = jnp.maximum(hmax_ref[...], h)

    @pl.when((i == 0) & (k == 0))
    def _init_s():
        s_ref[...] = s
        ss_ref[...] = ss

    @pl.when((i > 0) | (k > 0))
    def _acc_s():
        s_ref[...] = s_ref[...] + s
        ss_ref[...] = ss_ref[...] + ss


def _edge_finalize(gath_km, f, W, b):
    F = f.shape[1]
    C = W.shape[1]
    Cg = gath_km.shape[1]
    return pl.pallas_call(
        functools.partial(_edge_fin_body, F=F),
        grid=(NBLK, K),
        in_specs=[
            pl.BlockSpec((BLK, Cg), lambda i, k: (k * NBLK + i, 0)),
            pl.BlockSpec((BLK, F), lambda i, k: (i, 0)),
            pl.BlockSpec((2 * F, C), lambda i, k: (0, 0)),
            pl.BlockSpec((1, C), lambda i, k: (0, 0)),
        ],
        out_specs=[
            pl.BlockSpec((BLK, C), lambda i, k: (i, 0)),
            pl.BlockSpec((1, C), lambda i, k: (0, 0)),
            pl.BlockSpec((1, C), lambda i, k: (0, 0)),
        ],
        out_shape=[
            jax.ShapeDtypeStruct((N, C), jnp.float32),
            jax.ShapeDtypeStruct((1, C), jnp.float32),
            jax.ShapeDtypeStruct((1, C), jnp.float32),
        ],
    )(gath_km, f, W, b.reshape(1, C))


def _bn_apply_body(hmax_ref, s_ref, ss_ref, g_ref, be_ref, out_ref, *, cnt):
    m = s_ref[...] / cnt
    var = ss_ref[...] / cnt - m * m
    scale = g_ref[...] * lax.rsqrt(var + 1e-5)
    out_ref[...] = scale * (hmax_ref[...] - m) + be_ref[...]


def _bn_apply(hmax, s, ss, g, be, cnt):
    C = hmax.shape[1]
    return pl.pallas_call(
        functools.partial(_bn_apply_body, cnt=float(cnt)),
        grid=(NBLK,),
        in_specs=[
            pl.BlockSpec((BLK, C), lambda i: (i, 0)),
            pl.BlockSpec((1, C), lambda i: (0, 0)),
            pl.BlockSpec((1, C), lambda i: (0, 0)),
            pl.BlockSpec((1, C), lambda i: (0, 0)),
            pl.BlockSpec((1, C), lambda i: (0, 0)),
        ],
        out_specs=pl.BlockSpec((BLK, C), lambda i: (i, 0)),
        out_shape=jax.ShapeDtypeStruct((N, C), jnp.float32),
    )(hmax, s, ss, g.reshape(1, C), be.reshape(1, C))


def _fc_seg_body(xc_ref, wfc_ref, bfc_ref, bat_ref,
                 s_ref, ss_ref, seg_ref):
    i = pl.program_id(0)
    h = lax.dot_general(xc_ref[...], wfc_ref[...], (((1,), (0,)), ((), ())),
                        preferred_element_type=jnp.float32)
    h = jnp.maximum(h + bfc_ref[...], 0.0)
    s = jnp.sum(h, axis=0, keepdims=True)
    ss = jnp.sum(h * h, axis=0, keepdims=True)
    bat = bat_ref[...]
    segs = []
    for s_ in range(B):
        hm = jnp.where(bat == s_, h, -jnp.inf)
        segs.append(jnp.max(hm, axis=0, keepdims=True))
    seg = jnp.concatenate(segs, axis=0)

    @pl.when(i == 0)
    def _init():
        s_ref[...] = s
        ss_ref[...] = ss
        seg_ref[...] = seg

    @pl.when(i > 0)
    def _acc():
        s_ref[...] = s_ref[...] + s
        ss_ref[...] = ss_ref[...] + ss
        seg_ref[...] = jnp.maximum(seg_ref[...], seg)


def _fc_seg(xcat, wfc, bfc, batch):
    return pl.pallas_call(
        _fc_seg_body,
        grid=(NBLK,),
        in_specs=[
            pl.BlockSpec((BLK, 192), lambda i: (i, 0)),
            pl.BlockSpec((192, 256), lambda i: (0, 0)),
            pl.BlockSpec((1, 256), lambda i: (0, 0)),
            pl.BlockSpec((BLK, 1), lambda i: (i, 0)),
        ],
        out_specs=[
            pl.BlockSpec((1, 256), lambda i: (0, 0)),
            pl.BlockSpec((1, 256), lambda i: (0, 0)),
            pl.BlockSpec((B, 256), lambda i: (0, 0)),
        ],
        out_shape=[
            jax.ShapeDtypeStruct((1, 256), jnp.float32),
            jax.ShapeDtypeStruct((1, 256), jnp.float32),
            jax.ShapeDtypeStruct((B, 256), jnp.float32),
        ],
    )(xcat, wfc, bfc.reshape(1, 256), batch.reshape(N, 1))


def _head_body(s_ref, ss_ref, seg_ref, gfc_ref, befc_ref,
               wo1_ref, bo1_ref, go1_ref, beo1_ref,
               wo2_ref, bo2_ref, go2_ref, beo2_ref,
               wo3_ref, bo3_ref, out_ref):
    m = s_ref[...] / float(N)
    var = ss_ref[...] / float(N) - m * m
    p_raw = seg_ref[...]
    p = gfc_ref[...] * lax.rsqrt(var + 1e-5) * (p_raw - m) + befc_ref[...]

    def bn0(o, g, be):
        mm = jnp.mean(o, axis=0, keepdims=True)
        vv = jnp.mean(o * o, axis=0, keepdims=True) - mm * mm
        return g * lax.rsqrt(vv + 1e-5) * (o - mm) + be

    o = lax.dot_general(p, wo1_ref[...], (((1,), (0,)), ((), ())),
                        preferred_element_type=jnp.float32) + bo1_ref[...]
    o = bn0(jnp.maximum(o, 0.0), go1_ref[...], beo1_ref[...])
    o = lax.dot_general(o, wo2_ref[...], (((1,), (0,)), ((), ())),
                        preferred_element_type=jnp.float32) + bo2_ref[...]
    o = bn0(jnp.maximum(o, 0.0), go2_ref[...], beo2_ref[...])
    o = lax.dot_general(o, wo3_ref[...], (((1,), (0,)), ((), ())),
                        preferred_element_type=jnp.float32) + bo3_ref[...]
    mx = jnp.max(o, axis=1, keepdims=True)
    lse = jnp.log(jnp.sum(jnp.exp(o - mx), axis=1, keepdims=True))
    out_ref[...] = o - mx - lse


def _head(s, ss, seg, gfc, befc, wo1, bo1, go1, beo1, wo2, bo2, go2, beo2, wo3, bo3):
    full = lambda *shape: pl.BlockSpec(shape, lambda: tuple(0 for _ in shape))
    return pl.pallas_call(
        _head_body,
        in_specs=[
            full(1, 256), full(1, 256), full(B, 256),
            full(1, 256), full(1, 256),
            full(256, 128), full(1, 128), full(1, 128), full(1, 128),
            full(128, 64), full(1, 64), full(1, 64), full(1, 64),
            full(64, 40), full(1, 40),
        ],
        out_specs=full(B, 40),
        out_shape=jax.ShapeDtypeStruct((B, 40), jnp.float32),
    )(s, ss, seg, gfc.reshape(1, 256), befc.reshape(1, 256),
      wo1, bo1.reshape(1, 128), go1.reshape(1, 128), beo1.reshape(1, 128),
      wo2, bo2.reshape(1, 64), go2.reshape(1, 64), beo2.reshape(1, 64),
      wo3, bo3.reshape(1, 40))


def kernel(pos, x, batch, W1, b1, g1, be1, W2, b2, g2, be2, Wfc, bfc, gfc, befc,
           Wo1, bo1, go1, beo1, Wo2, bo2, go2, beo2, Wo3, bo3):
    batch = batch.astype(jnp.int32)
    f0 = jnp.concatenate([pos, x], axis=1)
    f0p8 = jnp.concatenate([f0, jnp.zeros((N, 4), jnp.float32)], axis=1)
    f0p128 = jnp.concatenate([f0, jnp.zeros((N, 124), jnp.float32)], axis=1)
    zeros_batch = jnp.zeros((N,), jnp.int32)

    idx1 = _knn(f0p8, zeros_batch)
    g1rows = _sc_gather(f0p128, idx1.T.reshape(N * K))
    hmax1, s1, ss1 = _edge_finalize(g1rows, f0, W1, b1)
    x1 = _bn_apply(hmax1, s1, ss1, g1, be1, N * K)

    idx2 = _knn(x1, batch)
    x1p128 = jnp.concatenate([x1, jnp.zeros((N, 64), jnp.float32)], axis=1)
    g2rows = _sc_gather(x1p128, idx2.T.reshape(N * K))
    hmax2, s2, ss2 = _edge_finalize(g2rows, x1, W2, b2)
    x2 = _bn_apply(hmax2, s2, ss2, g2, be2, N * K)

    sfc, ssfc, seg = _fc_seg(jnp.concatenate([x1, x2], axis=1), Wfc, bfc, batch)
    return _head(sfc, ssfc, seg, gfc, befc, Wo1, bo1, go1, beo1,
                 Wo2, bo2, go2, beo2, Wo3, bo3)
